# baseline (speedup 1.0000x reference)
import functools

import jax
import jax.numpy as jnp
from jax import lax
from jax.experimental import pallas as pl
from jax.experimental.pallas import tpu as pltpu
from jax.experimental.pallas import tpu_sc as plsc

N = 4000000
NW = 32
PER = N // NW
STEPS = 25
R = PER // STEPS
GW = 6 * R
VW = 3 * R
FULL_GROUPS = R // 16
REM = R - 16 * FULL_GROUPS


def _accum_group(gbuf, vbuf, accs, b6, b3, lane_ok):
    accs = list(accs)
    for j in range(3):
        bcf = plsc.load_gather(gbuf, [b6 + (2 * j)])
        dst = plsc.load_gather(gbuf, [b6 + (2 * j + 1)])
        val = plsc.load_gather(vbuf, [b3 + j])
        nanm = val != val
        if lane_ok is None:
            bcm = jnp.logical_not(nanm)
            nbcm = nanm
        else:
            bcm = jnp.logical_not(nanm) & lane_ok
            nbcm = nanm & lane_ok
        negm = (dst < 0.0) & nbcm
        d = bcf - val
        sq = d * d
        sqd = dst * dst
        k = 5 * j
        accs[k + 0] = accs[k + 0] + jnp.where(bcm, sq, 0.0)
        accs[k + 1] = accs[k + 1] + jnp.where(bcm, sqd, 0.0)
        accs[k + 2] = accs[k + 2] + jnp.where(bcm, 1.0, 0.0)
        accs[k + 3] = accs[k + 3] + jnp.where(negm, sqd, 0.0)
        accs[k + 4] = accs[k + 4] + jnp.where(negm, 1.0, 0.0)
    return tuple(accs)


def _process_chunk(gbuf, vbuf, accs):
    rowi = lax.iota(jnp.int32, 16)
    r6 = rowi * 6
    r3 = rowi * 3

    def grp(g, accs):
        b6 = r6 + g * 96
        b3 = r3 + g * 48
        return _accum_group(gbuf, vbuf, accs, b6, b3, None)

    accs = lax.fori_loop(0, FULL_GROUPS, grp, accs)
    b6 = jnp.minimum(r6 + FULL_GROUPS * 96, GW - 6)
    b3 = jnp.minimum(r3 + FULL_GROUPS * 48, VW - 3)
    lane_ok = rowi < REM
    return _accum_group(gbuf, vbuf, accs, b6, b3, lane_ok)


def _sc_body(gen_hbm, vals_hbm, out_hbm, gb0, gb1, vb0, vb1, ob,
             sg0, sg1, sv0, sv1):
    wid = lax.axis_index("s") * 2 + lax.axis_index("c")
    gbase = wid * (PER * 6)
    vbase = wid * (PER * 3)
    gbufs = (gb0, gb1)
    vbufs = (vb0, vb1)
    gsems = (sg0, sg1)
    vsems = (sv0, sv1)

    def gsrc(s):
        return gen_hbm.at[pl.ds(gbase + s * GW, GW)]

    def vsrc(s):
        return vals_hbm.at[pl.ds(vbase + s * VW, VW)]

    def start(s, b):
        pltpu.async_copy(gsrc(s), gbufs[b], gsems[b])
        pltpu.async_copy(vsrc(s), vbufs[b], vsems[b])

    def wait(s, b):
        pltpu.make_async_copy(gsrc(s), gbufs[b], gsems[b]).wait()
        pltpu.make_async_copy(vsrc(s), vbufs[b], vsems[b]).wait()

    accs = tuple(jnp.zeros((16,), jnp.float32) for _ in range(15))
    start(0, 0)

    def two_steps(i, accs):
        s0 = 2 * i
        start(s0 + 1, 1)
        wait(s0, 0)
        accs = _process_chunk(gb0, vb0, accs)
        start(s0 + 2, 0)
        wait(s0 + 1, 1)
        accs = _process_chunk(gb1, vb1, accs)
        return accs

    accs = lax.fori_loop(0, (STEPS - 1) // 2, two_steps, accs)
    wait(STEPS - 1, 0)
    accs = _process_chunk(gb0, vb0, accs)

    for i in range(15):
        ob[i] = accs[i]
    pltpu.sync_copy(ob, out_hbm.at[wid])


@jax.jit
def _sc_partials(gen_flat, vals_flat):
    mesh = plsc.VectorSubcoreMesh(core_axis_name="c", subcore_axis_name="s")
    f = pl.kernel(
        _sc_body,
        mesh=mesh,
        compiler_params=pltpu.CompilerParams(needs_layout_passes=False),
        out_type=jax.ShapeDtypeStruct((NW, 15, 16), jnp.float32),
        scratch_types=[
            pltpu.VMEM((GW,), jnp.float32),
            pltpu.VMEM((GW,), jnp.float32),
            pltpu.VMEM((VW,), jnp.float32),
            pltpu.VMEM((VW,), jnp.float32),
            pltpu.VMEM((15, 16), jnp.float32),
            pltpu.SemaphoreType.DMA,
            pltpu.SemaphoreType.DMA,
            pltpu.SemaphoreType.DMA,
            pltpu.SemaphoreType.DMA,
        ],
    )
    return f(gen_flat, vals_flat)


def kernel(generated_outputs, vals):
    gen_flat = generated_outputs.reshape(-1)
    vals_flat = vals.reshape(-1)
    parts = _sc_partials(gen_flat, vals_flat)
    t = parts.sum(axis=(0, 2)).reshape(3, 5)
    bc_cnt = jnp.maximum(t[:, 2], 1.0)
    neg_cnt = jnp.maximum(t[:, 4], 1.0)
    loss = jnp.sum((t[:, 0] + t[:, 1]) / bc_cnt + t[:, 3] / neg_cnt)
    return loss


# speedup vs baseline: 83.1956x; 83.1956x over previous
import functools

import jax
import jax.numpy as jnp
from jax.experimental import pallas as pl
from jax.experimental.pallas import tpu as pltpu


def _body(gen_ref, vals_ref, out_ref, *, bt):
    step = pl.program_id(0)
    g = gen_ref[...]
    v = vals_ref[...].reshape(3, bt, 128)
    bcf = g[:, :, 0, :]
    dist = g[:, :, 1, :]
    nanm = v != v
    d = bcf - v
    sq = jnp.where(nanm, 0.0, d * d)
    sqd = dist * dist
    sqd_bc = jnp.where(nanm, 0.0, sqd)
    bc_f = jnp.where(nanm, 0.0, 1.0)
    negm = nanm & (dist < 0.0)
    sq_neg = jnp.where(negm, sqd, 0.0)
    neg_f = jnp.where(negm, 1.0, 0.0)
    rows = [sq, sqd_bc, bc_f, sq_neg, neg_f]
    part = jnp.concatenate([jnp.sum(r, axis=1) for r in rows], axis=0)
    part = jnp.concatenate([part, jnp.zeros((1, 128), jnp.float32)], axis=0)

    @pl.when(step == 0)
    def _():
        out_ref[...] = jnp.zeros_like(out_ref)

    out_ref[...] += part


@jax.jit
def _partials(gen4, vals_t):
    nt = gen4.shape[1]
    bt = 250
    while nt % bt:
        bt //= 5
    grid = nt // bt
    f = pl.pallas_call(
        functools.partial(_body, bt=bt),
        grid=(grid,),
        in_specs=[
            pl.BlockSpec((3, bt, 2, 128), lambda i: (0, i, 0, 0)),
            pl.BlockSpec((3, bt * 128), lambda i: (0, i)),
        ],
        out_specs=pl.BlockSpec((16, 128), lambda i: (0, 0)),
        out_shape=jax.ShapeDtypeStruct((16, 128), jnp.float32),
    )
    return f(gen4, vals_t)


def kernel(generated_outputs, vals):
    n = generated_outputs.shape[0]
    gen4 = (
        generated_outputs.transpose(1, 0, 2)
        .reshape(3, n // 128, 128, 2)
        .transpose(0, 1, 3, 2)
    )
    vals_t = vals.transpose(1, 0)
    parts = _partials(gen4, vals_t)
    t = parts.sum(axis=1)
    sq, sqd, cnt = t[0:3], t[3:6], t[6:9]
    neg, ncnt = t[9:12], t[12:15]
    bc_cnt = jnp.maximum(cnt, 1.0)
    neg_cnt = jnp.maximum(ncnt, 1.0)
    return jnp.sum((sq + sqd) / bc_cnt + neg / neg_cnt)


# speedup vs baseline: 130.1563x; 1.5645x over previous
import functools

import jax
import jax.numpy as jnp
from jax.experimental import pallas as pl
from jax.experimental.pallas import tpu as pltpu


def _acc_slice(bcf, dist, v, accs):
    a1, a2, a3, a4 = accs
    nan = v != v
    w = jnp.where(nan, 1.0, 0.0)
    sel = jnp.where(nan, bcf, v)
    d = bcf - sel
    d2 = jnp.where(nan, 0.0, dist * dist)
    xm = jnp.minimum(dist, 0.0)
    sn = (xm * xm) * w
    nf = jnp.where(xm < 0.0, w, 0.0)
    return (a1 + (d * d + d2), a2 + sn, a3 + nf, a4 + w)


def _body(g0, g1, g2, v_ref, out_ref, *, bt):
    step = pl.program_id(0)
    gen = (g0, g1, g2)
    rows = []
    for j in range(3):
        zero = jnp.zeros((8, 128), jnp.float32)
        accs = (zero, zero, zero, zero)
        gref = gen[j]
        n8 = bt // 8
        for k in range(n8):
            sl = pl.ds(8 * k, 8)
            accs = _acc_slice(gref[0, sl, 0, :], gref[0, sl, 1, :],
                              v_ref[sl, j, :], accs)
        rem = bt - 8 * n8
        if rem:
            sl = pl.ds(8 * n8, rem)
            zr = jnp.zeros((rem, 128), jnp.float32)
            raccs = _acc_slice(gref[0, sl, 0, :], gref[0, sl, 1, :],
                               v_ref[sl, j, :], (zr, zr, zr, zr))
            accs = tuple(jnp.concatenate([a, r], axis=0)
                         for a, r in zip(accs, raccs))
        rows.append([jnp.sum(a, axis=0, keepdims=True) for a in accs])
    part = jnp.concatenate(
        [rows[j][q] for q in range(4) for j in range(3)]
        + [jnp.zeros((4, 128), jnp.float32)],
        axis=0,
    )

    @pl.when(step == 0)
    def _():
        out_ref[...] = jnp.zeros_like(out_ref)

    out_ref[...] += part


@jax.jit
def _partials(gen4, vals3):
    nt = gen4.shape[1]
    bt = 250
    while nt % bt:
        bt //= 5
    grid = nt // bt
    gen_specs = [
        pl.BlockSpec((1, bt, 2, 128), lambda i, j=j: (j, i, 0, 0))
        for j in range(3)
    ]
    val_spec = pl.BlockSpec((bt, 3, 128), lambda i: (i, 0, 0))
    f = pl.pallas_call(
        functools.partial(_body, bt=bt),
        grid=(grid,),
        in_specs=gen_specs + [val_spec],
        out_specs=pl.BlockSpec((16, 128), lambda i: (0, 0)),
        out_shape=jax.ShapeDtypeStruct((16, 128), jnp.float32),
    )
    return f(gen4, gen4, gen4, vals3)


def kernel(generated_outputs, vals):
    n = generated_outputs.shape[0]
    gen4 = (
        generated_outputs.transpose(1, 0, 2)
        .reshape(3, n // 128, 128, 2)
        .transpose(0, 1, 3, 2)
    )
    vals3 = vals.reshape(n // 128, 128, 3).transpose(0, 2, 1)
    parts = _partials(gen4, vals3)
    t = parts.sum(axis=1)
    s1, sn, ncnt, nanc = t[0:3], t[3:6], t[6:9], t[9:12]
    cnt = jnp.float32(n) - nanc
    return jnp.sum(s1 / jnp.maximum(cnt, 1.0) + sn / jnp.maximum(ncnt, 1.0))


# speedup vs baseline: 296.4465x; 2.2776x over previous
import functools

import jax
import jax.numpy as jnp
from jax.experimental import pallas as pl
from jax.experimental.pallas import tpu as pltpu

CH = 250


def _acc_slice(bcf, dist, v, accs):
    a1, a2, a3, a4 = accs
    nan = v != v
    w = jnp.where(nan, 1.0, 0.0)
    sel = jnp.where(nan, bcf, v)
    d = bcf - sel
    d2 = jnp.where(nan, 0.0, dist * dist)
    xm = jnp.minimum(dist, 0.0)
    sn = (xm * xm) * w
    nf = jnp.where(xm < 0.0, w, 0.0)
    return (a1 + (d * d + d2), a2 + sn, a3 + nf, a4 + w)


def _body(gen_hbm, vals_hbm, out_ref, buf, sems):
    nt = gen_hbm.shape[1]
    steps = nt // CH

    def copies(s, slot):
        t0 = s * CH
        cps = []
        for j in range(3):
            for c in range(2):
                cps.append(pltpu.make_async_copy(
                    gen_hbm.at[j, pl.ds(t0, CH), c, 0, :],
                    buf.at[slot, 2 * j + c],
                    sems.at[slot, 2 * j + c]))
        for j in range(3):
            cps.append(pltpu.make_async_copy(
                vals_hbm.at[pl.ds(t0, CH), j, :],
                buf.at[slot, 6 + j],
                sems.at[slot, 6 + j]))
        return cps

    def start(s, slot):
        for cp in copies(s, slot):
            cp.start()

    def wait(s, slot):
        for cp in copies(s, slot):
            cp.wait()

    def process(slot, accs):
        accs = list(accs)
        n8 = CH // 8
        for j in range(3):
            cj = tuple(accs[4 * j:4 * j + 4])
            for k in range(n8):
                sl = pl.ds(8 * k, 8)
                cj = _acc_slice(buf[slot, 2 * j, sl, :],
                                buf[slot, 2 * j + 1, sl, :],
                                buf[slot, 6 + j, sl, :], cj)
            rem = CH - 8 * n8
            if rem:
                sl = pl.ds(8 * n8, rem)
                zr = jnp.zeros((rem, 128), jnp.float32)
                rj = _acc_slice(buf[slot, 2 * j, sl, :],
                                buf[slot, 2 * j + 1, sl, :],
                                buf[slot, 6 + j, sl, :], (zr, zr, zr, zr))
                pad = jnp.zeros((8 - rem, 128), jnp.float32)
                cj = tuple(a + jnp.concatenate([r, pad], axis=0)
                           for a, r in zip(cj, rj))
            accs[4 * j:4 * j + 4] = list(cj)
        return tuple(accs)

    zero = jnp.zeros((8, 128), jnp.float32)
    accs = (zero,) * 12
    start(0, 0)

    def two_steps(i, accs):
        s0 = 2 * i
        start(s0 + 1, 1)
        wait(s0, 0)
        accs = process(0, accs)
        start(s0 + 2, 0)
        wait(s0 + 1, 1)
        accs = process(1, accs)
        return accs

    accs = jax.lax.fori_loop(0, (steps - 1) // 2, two_steps, accs)
    wait(steps - 1, 0)
    accs = process(0, accs)

    part = jnp.concatenate(
        [jnp.sum(accs[4 * j + q], axis=0, keepdims=True)
         for q in range(4) for j in range(3)]
        + [jnp.zeros((4, 128), jnp.float32)],
        axis=0,
    )
    out_ref[...] = part


@jax.jit
def _partials(gen5, vals3):
    f = pl.pallas_call(
        _body,
        in_specs=[
            pl.BlockSpec(memory_space=pl.ANY),
            pl.BlockSpec(memory_space=pl.ANY),
        ],
        out_specs=pl.BlockSpec((16, 128), lambda: (0, 0)),
        out_shape=jax.ShapeDtypeStruct((16, 128), jnp.float32),
        scratch_shapes=[
            pltpu.VMEM((2, 9, CH, 128), jnp.float32),
            pltpu.SemaphoreType.DMA((2, 9)),
        ],
    )
    return f(gen5, vals3)


def kernel(generated_outputs, vals):
    n = generated_outputs.shape[0]
    gen5 = (
        generated_outputs.transpose(1, 0, 2)
        .reshape(3, n // 128, 128, 2)
        .transpose(0, 1, 3, 2)
        .reshape(3, n // 128, 2, 1, 128)
    )
    vals3 = vals.reshape(n // 128, 128, 3).transpose(0, 2, 1)
    parts = _partials(gen5, vals3)
    t = parts.sum(axis=1)
    s1, sn, ncnt, nanc = t[0:3], t[3:6], t[6:9], t[9:12]
    cnt = jnp.float32(n) - nanc
    return jnp.sum(s1 / jnp.maximum(cnt, 1.0) + sn / jnp.maximum(ncnt, 1.0))


# speedup vs baseline: 479.8529x; 1.6187x over previous
import functools

import jax
import jax.numpy as jnp
from jax.experimental import pallas as pl
from jax.experimental.pallas import tpu as pltpu

CH = 625


def _acc_slice(bcf, dist, v, accs):
    a1, a2, a3, a4 = accs
    nan = v != v
    w = jnp.where(nan, 1.0, 0.0)
    sel = jnp.where(nan, bcf, v)
    d = bcf - sel
    d2 = jnp.where(nan, 0.0, dist * dist)
    xm = jnp.minimum(dist, 0.0)
    sn = (xm * xm) * w
    nf = jnp.where(xm < 0.0, w, 0.0)
    return (a1 + (d * d + d2), a2 + sn, a3 + nf, a4 + w)


def _body(gen_hbm, vals_hbm, out_ref, buf, sems):
    nt = gen_hbm.shape[1]
    steps = nt // CH

    def copies(s, slot):
        t0 = s * CH
        cps = []
        for j in range(3):
            for c in range(2):
                cps.append(pltpu.make_async_copy(
                    gen_hbm.at[j, pl.ds(t0, CH), c, 0, :],
                    buf.at[slot, 2 * j + c],
                    sems.at[slot, 2 * j + c]))
        for j in range(3):
            cps.append(pltpu.make_async_copy(
                vals_hbm.at[pl.ds(t0, CH), j, :],
                buf.at[slot, 6 + j],
                sems.at[slot, 6 + j]))
        return cps

    def start(s, slot):
        for cp in copies(s, slot):
            cp.start()

    def wait(s, slot):
        for cp in copies(s, slot):
            cp.wait()

    def process(slot, accs):
        accs = list(accs)
        n8 = CH // 8
        for j in range(3):
            cj = tuple(accs[4 * j:4 * j + 4])
            for k in range(n8):
                sl = pl.ds(8 * k, 8)
                cj = _acc_slice(buf[slot, 2 * j, sl, :],
                                buf[slot, 2 * j + 1, sl, :],
                                buf[slot, 6 + j, sl, :], cj)
            rem = CH - 8 * n8
            if rem:
                sl = pl.ds(8 * n8, rem)
                zr = jnp.zeros((rem, 128), jnp.float32)
                rj = _acc_slice(buf[slot, 2 * j, sl, :],
                                buf[slot, 2 * j + 1, sl, :],
                                buf[slot, 6 + j, sl, :], (zr, zr, zr, zr))
                pad = jnp.zeros((8 - rem, 128), jnp.float32)
                cj = tuple(a + jnp.concatenate([r, pad], axis=0)
                           for a, r in zip(cj, rj))
            accs[4 * j:4 * j + 4] = list(cj)
        return tuple(accs)

    zero = jnp.zeros((8, 128), jnp.float32)
    accs = (zero,) * 12
    start(0, 0)

    def two_steps(i, accs):
        s0 = 2 * i
        start(s0 + 1, 1)
        wait(s0, 0)
        accs = process(0, accs)
        start(s0 + 2, 0)
        wait(s0 + 1, 1)
        accs = process(1, accs)
        return accs

    accs = jax.lax.fori_loop(0, (steps - 1) // 2, two_steps, accs)
    wait(steps - 1, 0)
    accs = process(0, accs)

    part = jnp.concatenate(
        [jnp.sum(accs[4 * j + q], axis=0, keepdims=True)
         for q in range(4) for j in range(3)]
        + [jnp.zeros((4, 128), jnp.float32)],
        axis=0,
    )
    out_ref[...] = part


@jax.jit
def _partials(gen5, vals3):
    f = pl.pallas_call(
        _body,
        in_specs=[
            pl.BlockSpec(memory_space=pl.ANY),
            pl.BlockSpec(memory_space=pl.ANY),
        ],
        out_specs=pl.BlockSpec((16, 128), lambda: (0, 0)),
        out_shape=jax.ShapeDtypeStruct((16, 128), jnp.float32),
        scratch_shapes=[
            pltpu.VMEM((2, 9, CH, 128), jnp.float32),
            pltpu.SemaphoreType.DMA((2, 9)),
        ],
    )
    return f(gen5, vals3)


def kernel(generated_outputs, vals):
    n = generated_outputs.shape[0]
    gen5 = (
        generated_outputs.transpose(1, 0, 2)
        .reshape(3, n // 128, 128, 2)
        .transpose(0, 1, 3, 2)
        .reshape(3, n // 128, 2, 1, 128)
    )
    vals3 = vals.reshape(n // 128, 128, 3).transpose(0, 2, 1)
    parts = _partials(gen5, vals3)
    t = parts.sum(axis=1)
    s1, sn, ncnt, nanc = t[0:3], t[3:6], t[6:9], t[9:12]
    cnt = jnp.float32(n) - nanc
    return jnp.sum(s1 / jnp.maximum(cnt, 1.0) + sn / jnp.maximum(ncnt, 1.0))


# speedup vs baseline: 579.9370x; 1.2086x over previous
import functools

import jax
import jax.numpy as jnp
from jax.experimental import pallas as pl
from jax.experimental.pallas import tpu as pltpu

CH = 1250


def _acc_slice(bcf, dist, v, accs):
    a1, a2, a3, a4 = accs
    nan = v != v
    w = jnp.where(nan, 1.0, 0.0)
    sel = jnp.where(nan, bcf, v)
    d = bcf - sel
    d2 = jnp.where(nan, 0.0, dist * dist)
    xm = jnp.minimum(dist, 0.0)
    sn = (xm * xm) * w
    nf = jnp.where(xm < 0.0, w, 0.0)
    return (a1 + (d * d + d2), a2 + sn, a3 + nf, a4 + w)


def _body(gen_hbm, vals_hbm, out_ref, buf, sems):
    nt = gen_hbm.shape[1]
    steps = nt // CH

    def copies(s, slot):
        t0 = s * CH
        cps = []
        for j in range(3):
            for c in range(2):
                cps.append(pltpu.make_async_copy(
                    gen_hbm.at[j, pl.ds(t0, CH), c, 0, :],
                    buf.at[slot, 2 * j + c],
                    sems.at[slot, 2 * j + c]))
        for j in range(3):
            cps.append(pltpu.make_async_copy(
                vals_hbm.at[pl.ds(t0, CH), j, :],
                buf.at[slot, 6 + j],
                sems.at[slot, 6 + j]))
        return cps

    def start(s, slot):
        for cp in copies(s, slot):
            cp.start()

    def wait(s, slot):
        for cp in copies(s, slot):
            cp.wait()

    def process(slot, accs):
        accs = list(accs)
        n8 = CH // 8
        for j in range(3):
            cj = tuple(accs[4 * j:4 * j + 4])
            for k in range(n8):
                sl = pl.ds(8 * k, 8)
                cj = _acc_slice(buf[slot, 2 * j, sl, :],
                                buf[slot, 2 * j + 1, sl, :],
                                buf[slot, 6 + j, sl, :], cj)
            rem = CH - 8 * n8
            if rem:
                sl = pl.ds(8 * n8, rem)
                zr = jnp.zeros((rem, 128), jnp.float32)
                rj = _acc_slice(buf[slot, 2 * j, sl, :],
                                buf[slot, 2 * j + 1, sl, :],
                                buf[slot, 6 + j, sl, :], (zr, zr, zr, zr))
                pad = jnp.zeros((8 - rem, 128), jnp.float32)
                cj = tuple(a + jnp.concatenate([r, pad], axis=0)
                           for a, r in zip(cj, rj))
            accs[4 * j:4 * j + 4] = list(cj)
        return tuple(accs)

    zero = jnp.zeros((8, 128), jnp.float32)
    accs = (zero,) * 12
    start(0, 0)

    def two_steps(i, accs):
        s0 = 2 * i
        start(s0 + 1, 1)
        wait(s0, 0)
        accs = process(0, accs)
        start(s0 + 2, 0)
        wait(s0 + 1, 1)
        accs = process(1, accs)
        return accs

    accs = jax.lax.fori_loop(0, (steps - 1) // 2, two_steps, accs)
    wait(steps - 1, 0)
    accs = process(0, accs)

    part = jnp.concatenate(
        [jnp.sum(accs[4 * j + q], axis=0, keepdims=True)
         for q in range(4) for j in range(3)]
        + [jnp.zeros((4, 128), jnp.float32)],
        axis=0,
    )
    out_ref[...] = part


@jax.jit
def _partials(gen5, vals3):
    f = pl.pallas_call(
        _body,
        in_specs=[
            pl.BlockSpec(memory_space=pl.ANY),
            pl.BlockSpec(memory_space=pl.ANY),
        ],
        out_specs=pl.BlockSpec((16, 128), lambda: (0, 0)),
        out_shape=jax.ShapeDtypeStruct((16, 128), jnp.float32),
        scratch_shapes=[
            pltpu.VMEM((2, 9, CH, 128), jnp.float32),
            pltpu.SemaphoreType.DMA((2, 9)),
        ],
    )
    return f(gen5, vals3)


def kernel(generated_outputs, vals):
    n = generated_outputs.shape[0]
    gen5 = (
        generated_outputs.transpose(1, 0, 2)
        .reshape(3, n // 128, 128, 2)
        .transpose(0, 1, 3, 2)
        .reshape(3, n // 128, 2, 1, 128)
    )
    vals3 = vals.reshape(n // 128, 128, 3).transpose(0, 2, 1)
    parts = _partials(gen5, vals3)
    t = parts.sum(axis=1)
    s1, sn, ncnt, nanc = t[0:3], t[3:6], t[6:9], t[9:12]
    cnt = jnp.float32(n) - nanc
    return jnp.sum(s1 / jnp.maximum(cnt, 1.0) + sn / jnp.maximum(ncnt, 1.0))


# speedup vs baseline: 580.0158x; 1.0001x over previous
import jax
import jax.numpy as jnp
from jax.experimental import pallas as pl
from jax.experimental.pallas import tpu as pltpu

CH = 1250


def _acc_slice(bcf, dist, v, accs):
    a1, a2, a3, a4 = accs
    nan = v != v
    w = jnp.where(nan, 1.0, 0.0)
    sel = jnp.where(nan, bcf, v)
    d = bcf - sel
    d2 = jnp.where(nan, 0.0, dist * dist)
    xm = jnp.minimum(dist, 0.0)
    sn = (xm * xm) * w
    nf = jnp.where(xm < 0.0, w, 0.0)
    return (a1 + (d * d + d2), a2 + sn, a3 + nf, a4 + w)


def _body(gen_hbm, vals_hbm, out_ref, buf, sems):
    nt = gen_hbm.shape[1]
    steps = nt // CH

    def copies(s, slot):
        t0 = s * CH
        cps = []
        for j in range(3):
            for c in range(2):
                cps.append(pltpu.make_async_copy(
                    gen_hbm.at[j, pl.ds(t0, CH), c, 0, :],
                    buf.at[slot, 2 * j + c],
                    sems.at[slot, 2 * j + c]))
        for j in range(3):
            cps.append(pltpu.make_async_copy(
                vals_hbm.at[pl.ds(t0, CH), j, :],
                buf.at[slot, 6 + j],
                sems.at[slot, 6 + j]))
        return cps

    def start(s, slot):
        for cp in copies(s, slot):
            cp.start()

    def wait(s, slot):
        for cp in copies(s, slot):
            cp.wait()

    def process(slot, accs):
        accs = list(accs)
        n8 = CH // 8
        for j in range(3):
            cj = tuple(accs[4 * j:4 * j + 4])
            for k in range(n8):
                sl = pl.ds(8 * k, 8)
                cj = _acc_slice(buf[slot, 2 * j, sl, :],
                                buf[slot, 2 * j + 1, sl, :],
                                buf[slot, 6 + j, sl, :], cj)
            rem = CH - 8 * n8
            if rem:
                sl = pl.ds(8 * n8, rem)
                zr = jnp.zeros((rem, 128), jnp.float32)
                rj = _acc_slice(buf[slot, 2 * j, sl, :],
                                buf[slot, 2 * j + 1, sl, :],
                                buf[slot, 6 + j, sl, :], (zr, zr, zr, zr))
                pad = jnp.zeros((8 - rem, 128), jnp.float32)
                cj = tuple(a + jnp.concatenate([r, pad], axis=0)
                           for a, r in zip(cj, rj))
            accs[4 * j:4 * j + 4] = list(cj)
        return tuple(accs)

    zero = jnp.zeros((8, 128), jnp.float32)
    accs = (zero,) * 12
    start(0, 0)

    def two_steps(i, accs):
        s0 = 2 * i
        start(s0 + 1, 1)
        wait(s0, 0)
        accs = process(0, accs)
        start(s0 + 2, 0)
        wait(s0 + 1, 1)
        accs = process(1, accs)
        return accs

    accs = jax.lax.fori_loop(0, (steps - 1) // 2, two_steps, accs)
    wait(steps - 1, 0)
    accs = process(0, accs)

    part = jnp.concatenate(
        [jnp.sum(accs[4 * j + q], axis=0, keepdims=True)
         for q in range(4) for j in range(3)]
        + [jnp.zeros((4, 128), jnp.float32)],
        axis=0,
    )
    out_ref[...] = part


@jax.jit
def _partials(gen5, vals3):
    f = pl.pallas_call(
        _body,
        in_specs=[
            pl.BlockSpec(memory_space=pl.ANY),
            pl.BlockSpec(memory_space=pl.ANY),
        ],
        out_specs=pl.BlockSpec((16, 128), lambda: (0, 0)),
        out_shape=jax.ShapeDtypeStruct((16, 128), jnp.float32),
        scratch_shapes=[
            pltpu.VMEM((2, 9, CH, 128), jnp.float32),
            pltpu.SemaphoreType.DMA((2, 9)),
        ],
    )
    return f(gen5, vals3)


def kernel(generated_outputs, vals):
    n = generated_outputs.shape[0]
    gen5 = (
        generated_outputs.transpose(1, 0, 2)
        .reshape(3, n // 128, 128, 2)
        .transpose(0, 1, 3, 2)
        .reshape(3, n // 128, 2, 1, 128)
    )
    vals3 = vals.reshape(n // 128, 128, 3).transpose(0, 2, 1)
    parts = _partials(gen5, vals3)
    t = parts.sum(axis=1)
    s1, sn, ncnt, nanc = t[0:3], t[3:6], t[6:9], t[9:12]
    cnt = jnp.float32(n) - nanc
    return jnp.sum(s1 / jnp.maximum(cnt, 1.0) + sn / jnp.maximum(ncnt, 1.0))
